# baseline (speedup 1.0000x reference)
import functools
import math

import jax
import jax.numpy as jnp
from jax import lax
from jax.experimental import pallas as pl
from jax.experimental.pallas import tpu as pltpu
from jax.experimental.pallas import tpu_sc as plsc

_TOPK = 2
_NUM_EXPERTS = 64
_NUM_BINS = 10
_EXP_BASE = 2.0
_ALIGNMENT = 64
_LANES = 16


def _make_sc_call(n_tokens: int):
    start = float(math.ceil(_TOPK * n_tokens / _NUM_EXPERTS))
    stop = float(_TOPK * n_tokens)
    scale = stop - start
    wsum = float(2.0 ** _NUM_BINS - 1.0)

    mesh = plsc.VectorSubcoreMesh(core_axis_name="c", subcore_axis_name="s")

    @functools.partial(
        pl.kernel,
        mesh=mesh,
        out_type=jax.ShapeDtypeStruct((_LANES,), jnp.int32),
        scratch_types=[
            pltpu.VMEM((_LANES,), jnp.int32),
            pltpu.VMEM((_LANES,), jnp.int32),
        ],
        compiler_params=pltpu.CompilerParams(needs_layout_passes=False),
    )
    def sc_call(cap_hbm, out_hbm, cap_v, out_v):
        is_lead = (lax.axis_index("c") == 0) & (lax.axis_index("s") == 0)
        pltpu.sync_copy(cap_hbm, cap_v)
        cap = cap_v[...]
        lane = lax.iota(jnp.int32, _LANES)

        def gen_edges(pos):
            csum = ((jnp.int32(2) << pos) - 1).astype(jnp.float32) / jnp.float32(wsum)
            edges_f = jnp.float32(start) + jnp.float32(scale) * csum
            q = edges_f / jnp.float32(_ALIGNMENT)
            qi = q.astype(jnp.int32)
            qi = jnp.where(qi.astype(jnp.float32) < q, qi + 1, qi)
            return qi * jnp.int32(_ALIGNMENT)

        edges = gen_edges(lane)
        prev_edges = gen_edges(jnp.maximum(lane - 1, jnp.int32(0)))
        ge = (lane < _NUM_BINS) & (edges >= cap)
        prev_ge = (lane > 0) & (prev_edges >= cap)
        sel = (ge & jnp.logical_not(prev_ge)) | (
            (lane == _NUM_BINS - 1) & (edges < cap)
        )
        idx = jnp.where(sel, jnp.int32(0), jnp.int32(_LANES - 1))
        plsc.store_scatter(out_v.at[...], [idx], edges)

        @pl.when(is_lead)
        def _():
            pltpu.sync_copy(out_v, out_hbm)

    return sc_call


def kernel(gate_output, capacity):
    cap_vec = jnp.full((_LANES,), capacity, dtype=jnp.int32)
    out = _make_sc_call(gate_output.shape[0])(cap_vec)
    return out[0]


# speedup vs baseline: 1.1167x; 1.1167x over previous
import functools
import math

import jax
import jax.numpy as jnp
from jax import lax
from jax.experimental import pallas as pl
from jax.experimental.pallas import tpu as pltpu
from jax.experimental.pallas import tpu_sc as plsc

_TOPK = 2
_NUM_EXPERTS = 64
_NUM_BINS = 10
_EXP_BASE = 2.0
_ALIGNMENT = 64
_LANES = 16


def _make_sc_call(n_tokens: int):
    start = float(math.ceil(_TOPK * n_tokens / _NUM_EXPERTS))
    stop = float(_TOPK * n_tokens)
    scale = stop - start
    wsum = float(2.0 ** _NUM_BINS - 1.0)

    mesh = plsc.VectorSubcoreMesh(
        core_axis_name="c", subcore_axis_name="s", num_cores=1, num_subcores=1
    )

    @functools.partial(
        pl.kernel,
        mesh=mesh,
        out_type=jax.ShapeDtypeStruct((1,), jnp.int32),
        scratch_types=[
            pltpu.VMEM((_LANES,), jnp.int32),
            pltpu.VMEM((_LANES,), jnp.int32),
        ],
        compiler_params=pltpu.CompilerParams(needs_layout_passes=False),
    )
    def sc_call(cap_hbm, out_hbm, cap_v, out_v):
        pltpu.sync_copy(cap_hbm, cap_v)
        cap = cap_v[...]
        lane = lax.iota(jnp.int32, _LANES)

        def gen_edges(pos):
            csum = ((jnp.int32(2) << pos) - 1).astype(jnp.float32) / jnp.float32(wsum)
            edges_f = jnp.float32(start) + jnp.float32(scale) * csum
            q = edges_f / jnp.float32(_ALIGNMENT)
            qi = q.astype(jnp.int32)
            qi = jnp.where(qi.astype(jnp.float32) < q, qi + 1, qi)
            return qi * jnp.int32(_ALIGNMENT)

        edges = gen_edges(lane)
        prev_edges = gen_edges(jnp.maximum(lane - 1, jnp.int32(0)))
        ge = (lane < _NUM_BINS) & (edges >= cap)
        prev_ge = (lane > 0) & (prev_edges >= cap)
        sel = (ge & jnp.logical_not(prev_ge)) | (
            (lane == _NUM_BINS - 1) & (edges < cap)
        )
        idx = jnp.where(sel, jnp.int32(0), jnp.int32(_LANES - 1))
        plsc.store_scatter(out_v.at[...], [idx], edges)
        pltpu.sync_copy(out_v.at[pl.ds(0, 1)], out_hbm)

    return sc_call


def kernel(gate_output, capacity):
    cap_vec = jnp.full((_LANES,), capacity, dtype=jnp.int32)
    out = _make_sc_call(gate_output.shape[0])(cap_vec)
    return jnp.reshape(out, ())


# speedup vs baseline: 1.2243x; 1.0964x over previous
import functools
import math

import jax
import jax.numpy as jnp
from jax import lax
from jax.experimental import pallas as pl
from jax.experimental.pallas import tpu as pltpu
from jax.experimental.pallas import tpu_sc as plsc

_TOPK = 2
_NUM_EXPERTS = 64
_NUM_BINS = 10
_EXP_BASE = 2.0
_ALIGNMENT = 64


def _make_sc_call(n_tokens: int):
    start = float(math.ceil(_TOPK * n_tokens / _NUM_EXPERTS))
    stop = float(_TOPK * n_tokens)
    scale = stop - start
    inv_wsum = 1.0 / float(2.0 ** _NUM_BINS - 1.0)
    inv_align = 1.0 / float(_ALIGNMENT)

    mesh = plsc.ScalarSubcoreMesh(axis_name="c", num_cores=1)

    @functools.partial(
        pl.kernel,
        mesh=mesh,
        out_type=jax.ShapeDtypeStruct((1,), jnp.int32),
        scratch_types=[
            pltpu.SMEM((1,), jnp.int32),
            pltpu.SMEM((1,), jnp.int32),
        ],
        compiler_params=pltpu.CompilerParams(needs_layout_passes=False),
    )
    def sc_call(cap_hbm, out_hbm, cap_s, out_s):
        pltpu.sync_copy(cap_hbm, cap_s)
        cap = cap_s[0]

        def gen_edge(i):
            csum = jnp.float32((2 << i) - 1) * jnp.float32(inv_wsum)
            edge_f = jnp.float32(start) + jnp.float32(scale) * csum
            q = edge_f * jnp.float32(inv_align)
            qi = q.astype(jnp.int32)
            qi = jnp.where(qi.astype(jnp.float32) < q, qi + 1, qi)
            return qi * jnp.int32(_ALIGNMENT)

        r = gen_edge(_NUM_BINS - 1)
        for i in range(_NUM_BINS - 2, -1, -1):
            e = gen_edge(i)
            r = jnp.where(e >= cap, e, r)
        out_s[0] = r
        pltpu.sync_copy(out_s, out_hbm)

    return sc_call


def kernel(gate_output, capacity):
    cap_vec = jnp.reshape(jnp.asarray(capacity, dtype=jnp.int32), (1,))
    out = _make_sc_call(gate_output.shape[0])(cap_vec)
    return jnp.reshape(out, ())
